# baseline (speedup 1.0000x reference)
import jax
import jax.numpy as jnp
from jax import lax
from jax.experimental import pallas as pl
from jax.experimental.pallas import tpu as pltpu
from jax.experimental.pallas import tpu_sc as plsc

NN = 10000
EE = 320000
DD = 128
DH = DD // 2
DX = DH + 16
NC = 2
NS = 16
NP = 10016
RPT = 632
RPL = NP - RPT * (NS - 1)
CH = 128
NCHT = 157
NCHA = NCHT * NS
E2 = NCHA * CH


def _proj_body(f_ref, ad_ref, ae_ref, pd_ref, pe_ref):
    f = f_ref[...]
    pd_ref[...] = jnp.sum(f * ad_ref[...], axis=1, keepdims=True)
    pe_ref[...] = jnp.sum(f * ae_ref[...], axis=1, keepdims=True)


def _projections(feat, att_dst, att_edge):
    B = 80
    pd, pe = pl.pallas_call(
        _proj_body,
        grid=(NN // B,),
        in_specs=[
            pl.BlockSpec((B, DD), lambda i: (i, 0)),
            pl.BlockSpec((1, DD), lambda i: (0, 0)),
            pl.BlockSpec((1, DD), lambda i: (0, 0)),
        ],
        out_specs=[
            pl.BlockSpec((B, 1), lambda i: (i, 0)),
            pl.BlockSpec((B, 1), lambda i: (i, 0)),
        ],
        out_shape=[jax.ShapeDtypeStruct((NN, 1), jnp.float32)] * 2,
    )(feat, att_dst, att_edge)
    return pd[:, 0], pe[:, 0]


def _sc_body(feat2, idx4, pd, pe,
             acc_out,
             pdt, pet, idxb, dstc, w3buf, rows, wsum,
             shacc, semga, semgb, semia, semib):
    cid = lax.axis_index("c")
    sid = lax.axis_index("s")
    feath = feat2.at[cid]
    semg = (semga, semgb)
    semi = (semia, semib)

    zero16 = jnp.zeros((16,), jnp.float32)
    iota16 = jnp.arange(16, dtype=jnp.int32)
    colDH = jnp.full((16,), DH, jnp.int32)

    def _zrow(i, _):
        for g in range(DX // 16):
            wsum[i, pl.ds(g * 16, 16)] = zero16
        return 0
    lax.fori_loop(0, CH, _zrow, 0)
    r0 = sid * RPT

    @pl.when(sid < NS - 1)
    def _():
        off = 0
        for sz in (128, 128, 128, 128, 120):
            pltpu.sync_copy(wsum.at[pl.ds(0, sz)], shacc.at[pl.ds(r0 + off, sz)])
            off += sz

    @pl.when(sid == NS - 1)
    def _():
        off = 0
        for sz in (128, 128, 128, 128, 24):
            pltpu.sync_copy(wsum.at[pl.ds(0, sz)], shacc.at[pl.ds(r0 + off, sz)])
            off += sz

    pltpu.sync_copy(pd, pdt)
    pltpu.sync_copy(pe, pet)
    plsc.subcore_barrier()

    cix0 = sid * NCHT

    def step(k, par, do_fire):
        nxt = (par + 1) % 2
        for l in range(3):
            pltpu.make_async_copy(feath.at[pl.ds(0, CH)],
                                  rows.at[par, l], semg[par]).wait()
        if do_fire:
            pltpu.make_async_copy(idx4.at[cix0], idxb.at[nxt],
                                  semi[nxt]).wait()
            for l in range(3):
                pltpu.async_copy(feath.at[idxb.at[nxt, 1 + l]],
                                 rows.at[nxt, l], semg[nxt])

        def _sg(g, _):
            s = pl.ds(g * 16, 16)
            dv = idxb[par, 0, s]
            m0 = idxb[par, 1, s]
            m1 = idxb[par, 2, s]
            m2 = idxb[par, 3, s]
            pdv = plsc.load_gather(pdt, [dv])
            p0 = plsc.load_gather(pet, [m0])
            p1 = plsc.load_gather(pet, [m1])
            p2 = plsc.load_gather(pet, [m2])
            sc = pdv + (p0 + p1 + p2) * (1.0 / 3.0)
            sc = jnp.where(sc >= 0.0, sc, sc * 0.2)
            w3 = jnp.exp(sc) * (1.0 / 3.0)
            w3buf[s] = w3
            dstc[s] = dv
            return 0
        lax.fori_loop(0, CH // 16, _sg, 0)

        if do_fire:
            @pl.when(k + 2 <= NCHT - 1)
            def _():
                pltpu.async_copy(idx4.at[cix0 + k + 2], idxb.at[par],
                                 semi[par])

        def _comb(i16, _):
            wv = w3buf[pl.ds(i16 * 16, 16)]
            plsc.store_scatter(wsum, [i16 * 16 + iota16, colDH], wv)
            for j in range(16):
                i = i16 * 16 + j
                w3 = wv[j]
                for g in range(DH // 16):
                    s = pl.ds(g * 16, 16)
                    v = rows[par, 0, i, s] + rows[par, 1, i, s] + rows[par, 2, i, s]
                    wsum[i, s] = v * w3
            return 0
        lax.fori_loop(0, CH // 16, _comb, 0)

        pltpu.sync_copy(wsum, shacc.at[dstc], add=True)

    pltpu.sync_copy(idx4.at[cix0], idxb.at[0])
    for l in range(3):
        pltpu.async_copy(feath.at[idxb.at[0, 1 + l]], rows.at[0, l], semga)
    pltpu.async_copy(idx4.at[cix0 + 1], idxb.at[1], semib)

    def _pair(j, _):
        step(j * 2, 0, True)
        step(j * 2 + 1, 1, True)
        return 0
    lax.fori_loop(0, (NCHT - 1) // 2, _pair, 0)
    step(NCHT - 1, 0, False)

    plsc.subcore_barrier()

    @pl.when(sid < NS - 1)
    def _():
        pltpu.sync_copy(shacc.at[pl.ds(r0, RPT)], acc_out.at[cid, pl.ds(r0, RPT)])

    @pl.when(sid == NS - 1)
    def _():
        pltpu.sync_copy(shacc.at[pl.ds(r0, RPL)], acc_out.at[cid, pl.ds(r0, RPL)])


def _sc_call(feat2, idx4, pd, pe):
    f32 = jnp.float32
    i32 = jnp.int32
    return pl.kernel(
        _sc_body,
        out_type=jax.ShapeDtypeStruct((NC, NP, DX), f32),
        mesh=plsc.VectorSubcoreMesh(core_axis_name="c", subcore_axis_name="s"),
        compiler_params=pltpu.CompilerParams(
            needs_layout_passes=False, use_tc_tiling_on_sc=False),
        scratch_types=[
            pltpu.VMEM((NP,), f32),
            pltpu.VMEM((NP,), f32),
            pltpu.VMEM((2, 4, CH), i32),
            pltpu.VMEM((CH,), i32),
            pltpu.VMEM((CH,), f32),
            pltpu.VMEM((2, 3, CH, DH), f32),
            pltpu.VMEM((CH, DX), f32),
            pltpu.VMEM_SHARED((NP, DX), f32),
            pltpu.SemaphoreType.DMA,
            pltpu.SemaphoreType.DMA,
            pltpu.SemaphoreType.DMA,
            pltpu.SemaphoreType.DMA,
        ],
    )(feat2, idx4, pd, pe)


def _fin_body(a_ref, o_ref):
    a = a_ref[...]
    lo = a[0, :, :DH] / (a[0, :, DH:DH + 1] * 3.0 + 1e-16)
    hi = a[1, :, :DH] / (a[1, :, DH:DH + 1] * 3.0 + 1e-16)
    o_ref[...] = jnp.concatenate([lo, hi], axis=1)


def _finalize(acc):
    B = 32
    return pl.pallas_call(
        _fin_body,
        grid=(NP // B,),
        in_specs=[pl.BlockSpec((2, B, DX), lambda i: (0, i, 0))],
        out_specs=pl.BlockSpec((B, DD), lambda i: (i, 0)),
        out_shape=jax.ShapeDtypeStruct((NP, DD), jnp.float32),
    )(acc)


def kernel(node_features, edge_index, metapath_idx, att_dst, att_edge):
    f32 = jnp.float32
    i32 = jnp.int32
    feat = node_features.astype(f32)
    dst = edge_index[1].astype(i32)
    mp = metapath_idx.astype(i32)
    pad = E2 - EE
    dstp = jnp.concatenate([dst, jnp.full((pad,), NN, i32)])
    mp0p = jnp.concatenate([mp[:, 0], jnp.zeros((pad,), i32)])
    mp1p = jnp.concatenate([mp[:, 1], jnp.zeros((pad,), i32)])
    mp2p = jnp.concatenate([mp[:, 2], jnp.zeros((pad,), i32)])
    idx4 = jnp.stack([dstp, mp0p, mp1p, mp2p], 0)
    idx4 = idx4.reshape(4, NCHA, CH).transpose(1, 0, 2)
    feat2 = jnp.stack([feat[:, :DH], feat[:, DH:]])

    pd, pe = _projections(feat, att_dst.astype(f32), att_edge.astype(f32))
    pdp = jnp.pad(pd, (0, NP - NN))
    pep = jnp.pad(pe, (0, NP - NN))

    acc = _sc_call(feat2, idx4, pdp, pep)
    out = _finalize(acc)
    return out[:NN]


# speedup vs baseline: 1.3654x; 1.3654x over previous
import jax
import jax.numpy as jnp
from jax import lax
from jax.experimental import pallas as pl
from jax.experimental.pallas import tpu as pltpu
from jax.experimental.pallas import tpu_sc as plsc

NN = 10000
EE = 320000
DD = 128
DH = DD // 2
NC = 2
NS = 16
NP = 10112
RPT = NP // NS
CH = 128
NCHT = 157
NCHA = NCHT * NS
E2 = NCHA * CH


def _proj_body(f_ref, ad_ref, ae_ref, pd_ref, pe_ref):
    f = f_ref[...]
    pd_ref[...] = jnp.sum(f * ad_ref[...], axis=1, keepdims=True)
    pe_ref[...] = jnp.sum(f * ae_ref[...], axis=1, keepdims=True)


def _projections(feat, att_dst, att_edge):
    B = 80
    pd, pe = pl.pallas_call(
        _proj_body,
        grid=(NN // B,),
        in_specs=[
            pl.BlockSpec((B, DD), lambda i: (i, 0)),
            pl.BlockSpec((1, DD), lambda i: (0, 0)),
            pl.BlockSpec((1, DD), lambda i: (0, 0)),
        ],
        out_specs=[
            pl.BlockSpec((B, 1), lambda i: (i, 0)),
            pl.BlockSpec((B, 1), lambda i: (i, 0)),
        ],
        out_shape=[jax.ShapeDtypeStruct((NN, 1), jnp.float32)] * 2,
    )(feat, att_dst, att_edge)
    return pd[:, 0], pe[:, 0]


def _sc_body(feat2, idx4, pd, pe,
             acc_out, den_out,
             pdt, pet, idxb, dstc, wbuf, w3buf, rows, wsum, dstage,
             shacc, shden, semga, semgb, semia, semib):
    cid = lax.axis_index("c")
    sid = lax.axis_index("s")
    feath = feat2.at[cid]
    semg = (semga, semgb)
    semi = (semia, semib)

    zero16 = jnp.zeros((16,), jnp.float32)

    def _zrow(i, _):
        for g in range(DH // 16):
            wsum[i, pl.ds(g * 16, 16)] = zero16
        return 0
    lax.fori_loop(0, CH, _zrow, 0)
    for g in range(8):
        wbuf[pl.ds(g * 16, 16)] = zero16
    r0 = sid * RPT
    off = 0
    for sz in (128, 128, 128, 128, 120):
        pltpu.sync_copy(wsum.at[pl.ds(0, sz)], shacc.at[pl.ds(r0 + off, sz)])
        pltpu.sync_copy(wbuf.at[pl.ds(0, sz)], shden.at[pl.ds(r0 + off, sz)])
        off += sz

    pltpu.sync_copy(pd, pdt)
    pltpu.sync_copy(pe, pet)
    plsc.subcore_barrier()

    cix0 = sid * NCHT

    def step(k, par, do_fire):
        nxt = (par + 1) % 2
        for l in range(3):
            pltpu.make_async_copy(feath.at[pl.ds(0, CH)],
                                  rows.at[par, l], semg[par]).wait()
        if do_fire:
            pltpu.make_async_copy(idx4.at[cix0], idxb.at[nxt],
                                  semi[nxt]).wait()
            for l in range(3):
                pltpu.async_copy(feath.at[idxb.at[nxt, 1 + l]],
                                 rows.at[nxt, l], semg[nxt])

        def _sg(g, _):
            s = pl.ds(g * 16, 16)
            dv = idxb[par, 0, s]
            m0 = idxb[par, 1, s]
            m1 = idxb[par, 2, s]
            m2 = idxb[par, 3, s]
            pdv = plsc.load_gather(pdt, [dv])
            p0 = plsc.load_gather(pet, [m0])
            p1 = plsc.load_gather(pet, [m1])
            p2 = plsc.load_gather(pet, [m2])
            sc = pdv + (p0 + p1 + p2) * (1.0 / 3.0)
            sc = jnp.where(sc >= 0.0, sc, sc * 0.2)
            w = jnp.exp(sc)
            wbuf[s] = w
            w3buf[s] = w * (1.0 / 3.0)
            dstc[s] = dv
            return 0
        lax.fori_loop(0, CH // 16, _sg, 0)

        if do_fire:
            @pl.when(k + 2 <= NCHT - 1)
            def _():
                pltpu.async_copy(idx4.at[cix0 + k + 2], idxb.at[par],
                                 semi[par])

        def _comb(i16, _):
            wv = w3buf[pl.ds(i16 * 16, 16)]
            for j in range(16):
                i = i16 * 16 + j
                w3 = wv[j]
                for g in range(DH // 16):
                    s = pl.ds(g * 16, 16)
                    v = rows[par, 0, i, s] + rows[par, 1, i, s] + rows[par, 2, i, s]
                    wsum[i, s] = v * w3
            return 0
        lax.fori_loop(0, CH // 16, _comb, 0)

        pltpu.sync_copy(wsum, shacc.at[dstc], add=True)
        pltpu.sync_copy(wbuf, shden.at[dstc], add=True)

    pltpu.sync_copy(idx4.at[cix0], idxb.at[0])
    for l in range(3):
        pltpu.async_copy(feath.at[idxb.at[0, 1 + l]], rows.at[0, l], semga)
    pltpu.async_copy(idx4.at[cix0 + 1], idxb.at[1], semib)

    def _pair(j, _):
        step(j * 2, 0, True)
        step(j * 2 + 1, 1, True)
        return 0
    lax.fori_loop(0, (NCHT - 1) // 2, _pair, 0)
    step(NCHT - 1, 0, False)

    plsc.subcore_barrier()
    pltpu.sync_copy(shacc.at[pl.ds(r0, RPT)], acc_out.at[cid, pl.ds(r0, RPT)])
    pltpu.sync_copy(shden.at[pl.ds(r0, RPT)], dstage)
    pltpu.sync_copy(dstage, den_out.at[pl.ds(cid * NP + r0, RPT)])


def _sc_call(feat2, idx4, pd, pe):
    f32 = jnp.float32
    i32 = jnp.int32
    return pl.kernel(
        _sc_body,
        out_type=[
            jax.ShapeDtypeStruct((NC, NP, DH), f32),
            jax.ShapeDtypeStruct((NC * NP,), f32),
        ],
        mesh=plsc.VectorSubcoreMesh(core_axis_name="c", subcore_axis_name="s"),
        compiler_params=pltpu.CompilerParams(
            needs_layout_passes=False, use_tc_tiling_on_sc=False),
        scratch_types=[
            pltpu.VMEM((NP,), f32),
            pltpu.VMEM((NP,), f32),
            pltpu.VMEM((2, 4, CH), i32),
            pltpu.VMEM((CH,), i32),
            pltpu.VMEM((CH,), f32),
            pltpu.VMEM((CH,), f32),
            pltpu.VMEM((2, 3, CH, DH), f32),
            pltpu.VMEM((CH, DH), f32),
            pltpu.VMEM((RPT,), f32),
            pltpu.VMEM_SHARED((NP, DH), f32),
            pltpu.VMEM_SHARED((NP,), f32),
            pltpu.SemaphoreType.DMA,
            pltpu.SemaphoreType.DMA,
            pltpu.SemaphoreType.DMA,
            pltpu.SemaphoreType.DMA,
        ],
    )(feat2, idx4, pd, pe)


def _fin_body(a_ref, d_ref, o_ref):
    lo = a_ref[0] / (d_ref[0] + 1e-16)
    hi = a_ref[1] / (d_ref[1] + 1e-16)
    o_ref[...] = jnp.concatenate([lo, hi], axis=1)


def _finalize(acc, den):
    B = 128
    return pl.pallas_call(
        _fin_body,
        grid=(NP // B,),
        in_specs=[
            pl.BlockSpec((2, B, DH), lambda i: (0, i, 0)),
            pl.BlockSpec((2, B, 1), lambda i: (0, i, 0)),
        ],
        out_specs=pl.BlockSpec((B, DD), lambda i: (i, 0)),
        out_shape=jax.ShapeDtypeStruct((NP, DD), jnp.float32),
    )(acc, den)


def kernel(node_features, edge_index, metapath_idx, att_dst, att_edge):
    f32 = jnp.float32
    i32 = jnp.int32
    feat = node_features.astype(f32)
    dst = edge_index[1].astype(i32)
    mp = metapath_idx.astype(i32)
    pad = E2 - EE
    dstp = jnp.concatenate([dst, jnp.full((pad,), NN, i32)])
    mp0p = jnp.concatenate([mp[:, 0], jnp.zeros((pad,), i32)])
    mp1p = jnp.concatenate([mp[:, 1], jnp.zeros((pad,), i32)])
    mp2p = jnp.concatenate([mp[:, 2], jnp.zeros((pad,), i32)])
    idx4 = jnp.stack([dstp, mp0p, mp1p, mp2p], 0)
    idx4 = idx4.reshape(4, NCHA, CH).transpose(1, 0, 2)
    feat2 = jnp.stack([feat[:, :DH], feat[:, DH:]])

    pd, pe = _projections(feat, att_dst.astype(f32), att_edge.astype(f32))
    pdp = jnp.pad(pd, (0, NP - NN))
    pep = jnp.pad(pe, (0, NP - NN))

    acc, den = _sc_call(feat2, idx4, pdp, pep)
    out = _finalize(acc, den.reshape(NC, NP, 1))
    return out[:NN]


# speedup vs baseline: 1.5550x; 1.1389x over previous
import jax
import jax.numpy as jnp
from jax import lax
from jax.experimental import pallas as pl
from jax.experimental.pallas import tpu as pltpu
from jax.experimental.pallas import tpu_sc as plsc

NN = 10000
EE = 320000
DD = 128
DH = DD // 2
NC = 2
NS = 16
NP = 10112
RPT = NP // NS
CH = 128
NCHT = 157
NCHA = NCHT * NS
E2 = NCHA * CH


def _proj_body(f_ref, ad_ref, ae_ref, pd_ref, pe_ref):
    f = f_ref[...]
    pd_ref[...] = jnp.sum(f * ad_ref[...], axis=1, keepdims=True)
    pe_ref[...] = jnp.sum(f * ae_ref[...], axis=1, keepdims=True)


def _projections(feat, att_dst, att_edge):
    B = 80
    pd, pe = pl.pallas_call(
        _proj_body,
        grid=(NN // B,),
        in_specs=[
            pl.BlockSpec((B, DD), lambda i: (i, 0)),
            pl.BlockSpec((1, DD), lambda i: (0, 0)),
            pl.BlockSpec((1, DD), lambda i: (0, 0)),
        ],
        out_specs=[
            pl.BlockSpec((B, 1), lambda i: (i, 0)),
            pl.BlockSpec((B, 1), lambda i: (i, 0)),
        ],
        out_shape=[jax.ShapeDtypeStruct((NN, 1), jnp.float32)] * 2,
    )(feat, att_dst, att_edge)
    return pd[:, 0], pe[:, 0]


def _sc_body(feat2, idx4, pd, pe,
             acc_out, den_out,
             pdt, pet, idxb, dstc, wbuf, w3buf, rows, wsum, dstage,
             shacc, shden, semga, semgb, semia, semib):
    cid = lax.axis_index("c")
    sid = lax.axis_index("s")
    feath = feat2.at[cid]
    semg = (semga, semgb)
    semi = (semia, semib)

    zero16 = jnp.zeros((16,), jnp.float32)

    def _zrow(i, _):
        for g in range(DH // 16):
            wsum[i, pl.ds(g * 16, 16)] = zero16
        return 0
    lax.fori_loop(0, CH, _zrow, 0)
    for g in range(8):
        wbuf[pl.ds(g * 16, 16)] = zero16
    r0 = sid * RPT
    off = 0
    for sz in (128, 128, 128, 128, 120):
        pltpu.sync_copy(wsum.at[pl.ds(0, sz)], shacc.at[pl.ds(r0 + off, sz)])
        pltpu.sync_copy(wbuf.at[pl.ds(0, sz)], shden.at[pl.ds(r0 + off, sz)])
        off += sz

    pltpu.sync_copy(pd, pdt)
    pltpu.sync_copy(pe, pet)
    plsc.subcore_barrier()

    cix0 = sid * NCHT

    def step(k, par, do_fire):
        nxt = (par + 1) % 2
        for l in range(3):
            pltpu.make_async_copy(feath.at[pl.ds(0, CH)],
                                  rows.at[par, l], semg[par]).wait()
        if do_fire:
            pltpu.make_async_copy(idx4.at[cix0], idxb.at[nxt],
                                  semi[nxt]).wait()
            for l in range(3):
                pltpu.async_copy(feath.at[idxb.at[nxt, 1 + l]],
                                 rows.at[nxt, l], semg[nxt])

        def _sg(g, _):
            s = pl.ds(g * 16, 16)
            dv = idxb[par, 0, s]
            m0 = idxb[par, 1, s]
            m1 = idxb[par, 2, s]
            m2 = idxb[par, 3, s]
            pdv = plsc.load_gather(pdt, [dv])
            p0 = plsc.load_gather(pet, [m0])
            p1 = plsc.load_gather(pet, [m1])
            p2 = plsc.load_gather(pet, [m2])
            sc = pdv + (p0 + p1 + p2) * (1.0 / 3.0)
            sc = jnp.where(sc >= 0.0, sc, sc * 0.2)
            w = jnp.exp(sc)
            wbuf[s] = w
            w3buf[s] = w * (1.0 / 3.0)
            dstc[s] = dv
            return 0
        lax.fori_loop(0, CH // 16, _sg, 0)

        if do_fire:
            @pl.when(k + 2 <= NCHT - 1)
            def _():
                pltpu.async_copy(idx4.at[cix0 + k + 2], idxb.at[par],
                                 semi[par])

        def _comb(i16, _):
            wv = w3buf[pl.ds(i16 * 16, 16)]
            for j in range(16):
                i = i16 * 16 + j
                w3 = wv[j]
                for g in range(DH // 16):
                    s = pl.ds(g * 16, 16)
                    v = rows[par, 0, i, s] + rows[par, 1, i, s] + rows[par, 2, i, s]
                    wsum[i, s] = v * w3
            return 0
        lax.fori_loop(0, CH // 16, _comb, 0, unroll=2)

        pltpu.sync_copy(wsum, shacc.at[dstc], add=True)
        pltpu.sync_copy(wbuf, shden.at[dstc], add=True)

    pltpu.sync_copy(idx4.at[cix0], idxb.at[0])
    for l in range(3):
        pltpu.async_copy(feath.at[idxb.at[0, 1 + l]], rows.at[0, l], semga)
    pltpu.async_copy(idx4.at[cix0 + 1], idxb.at[1], semib)

    def _pair(j, _):
        step(j * 2, 0, True)
        step(j * 2 + 1, 1, True)
        return 0
    lax.fori_loop(0, (NCHT - 1) // 2, _pair, 0)
    step(NCHT - 1, 0, False)

    plsc.subcore_barrier()
    pltpu.sync_copy(shacc.at[pl.ds(r0, RPT)], acc_out.at[cid, pl.ds(r0, RPT)])
    pltpu.sync_copy(shden.at[pl.ds(r0, RPT)], dstage)
    pltpu.sync_copy(dstage, den_out.at[pl.ds(cid * NP + r0, RPT)])


def _sc_call(feat2, idx4, pd, pe):
    f32 = jnp.float32
    i32 = jnp.int32
    return pl.kernel(
        _sc_body,
        out_type=[
            jax.ShapeDtypeStruct((NC, NP, DH), f32),
            jax.ShapeDtypeStruct((NC * NP,), f32),
        ],
        mesh=plsc.VectorSubcoreMesh(core_axis_name="c", subcore_axis_name="s"),
        compiler_params=pltpu.CompilerParams(
            needs_layout_passes=False, use_tc_tiling_on_sc=False),
        scratch_types=[
            pltpu.VMEM((NP,), f32),
            pltpu.VMEM((NP,), f32),
            pltpu.VMEM((2, 4, CH), i32),
            pltpu.VMEM((CH,), i32),
            pltpu.VMEM((CH,), f32),
            pltpu.VMEM((CH,), f32),
            pltpu.VMEM((2, 3, CH, DH), f32),
            pltpu.VMEM((CH, DH), f32),
            pltpu.VMEM((RPT,), f32),
            pltpu.VMEM_SHARED((NP, DH), f32),
            pltpu.VMEM_SHARED((NP,), f32),
            pltpu.SemaphoreType.DMA,
            pltpu.SemaphoreType.DMA,
            pltpu.SemaphoreType.DMA,
            pltpu.SemaphoreType.DMA,
        ],
    )(feat2, idx4, pd, pe)


def _fin_body(a_ref, d_ref, o_ref):
    lo = a_ref[0] / (d_ref[0] + 1e-16)
    hi = a_ref[1] / (d_ref[1] + 1e-16)
    o_ref[...] = jnp.concatenate([lo, hi], axis=1)


def _finalize(acc, den):
    B = 128
    return pl.pallas_call(
        _fin_body,
        grid=(NP // B,),
        in_specs=[
            pl.BlockSpec((2, B, DH), lambda i: (0, i, 0)),
            pl.BlockSpec((2, B, 1), lambda i: (0, i, 0)),
        ],
        out_specs=pl.BlockSpec((B, DD), lambda i: (i, 0)),
        out_shape=jax.ShapeDtypeStruct((NP, DD), jnp.float32),
    )(acc, den)


def kernel(node_features, edge_index, metapath_idx, att_dst, att_edge):
    f32 = jnp.float32
    i32 = jnp.int32
    feat = node_features.astype(f32)
    dst = edge_index[1].astype(i32)
    mp = metapath_idx.astype(i32)
    pad = E2 - EE
    dstp = jnp.concatenate([dst, jnp.full((pad,), NN, i32)])
    mp0p = jnp.concatenate([mp[:, 0], jnp.zeros((pad,), i32)])
    mp1p = jnp.concatenate([mp[:, 1], jnp.zeros((pad,), i32)])
    mp2p = jnp.concatenate([mp[:, 2], jnp.zeros((pad,), i32)])
    idx4 = jnp.stack([dstp, mp0p, mp1p, mp2p], 0)
    idx4 = idx4.reshape(4, NCHA, CH).transpose(1, 0, 2)
    feat2 = jnp.stack([feat[:, :DH], feat[:, DH:]])

    pd, pe = _projections(feat, att_dst.astype(f32), att_edge.astype(f32))
    pdp = jnp.pad(pd, (0, NP - NN))
    pep = jnp.pad(pe, (0, NP - NN))

    acc, den = _sc_call(feat2, idx4, pdp, pep)
    out = _finalize(acc, den.reshape(NC, NP, 1))
    return out[:NN]
